# Initial kernel scaffold; baseline (speedup 1.0000x reference)
import jax
import jax.numpy as jnp
from jax.experimental import pallas as pl


def kernel(x, f, W_it, b_it, W_l1, gn_g, gn_b, W_m1a, b_m1a, W_m1b, b_m1b, W_m2a, b_m2a, W_m2b, b_m2b, W_m3a, b_m3a, W_m3b, b_m3b):
    raise NotImplementedError("write your pallas kernel here")


# speedup vs baseline: 7.9027x; 7.9027x over previous
import functools

import jax
import jax.numpy as jnp
from jax import lax
from jax.experimental import pallas as pl
from jax.experimental.pallas import tpu as pltpu
from jax.experimental.pallas import tpu_sc as plsc

K = 16
EPS = 1e-5


def _norms_body(x_ref, out_ref):
    x0 = x_ref[0, 0:1, :]
    x1 = x_ref[0, 1:2, :]
    x2 = x_ref[0, 2:3, :]
    out_ref[0] = (x0 * x0 + x1 * x1) + x2 * x2


def _f1_body(fT_ref, w_ref, b_ref, wl1T_ref, out_ref, tab_ref):
    f1 = (
        jnp.dot(fT_ref[0], w_ref[...], preferred_element_type=jnp.float32)
        + b_ref[...]
    )
    out_ref[0] = f1
    tab_ref[0] = jnp.dot(
        f1, wl1T_ref[:64, :], preferred_element_type=jnp.float32
    )


def _topk_body(rows_ref, full_ref, fn_ref, out_ref, *, n_total, rows):
    b = pl.program_id(0)
    xr = rows_ref[0]
    xf = full_ref[0]
    ab = lax.dot_general(
        xr, xf, (((1,), (1,)), ((), ())), preferred_element_type=jnp.float32
    )
    c0 = xr[:, 0:1]
    c1 = xr[:, 1:2]
    c2 = xr[:, 2:3]
    rn = (c0 * c0 + c1 * c1) + c2 * c2
    fn = fn_ref[0]
    d = -2.0 * ab + rn + fn

    iota = lax.broadcasted_iota(jnp.int32, (rows, n_total), 1)
    picks = []
    big = jnp.float32(jnp.inf)
    for _ in range(K):
        m = jnp.min(d, axis=1, keepdims=True)
        am = jnp.min(
            jnp.where(d == m, iota, n_total), axis=1, keepdims=True
        )
        picks.append(am)
        d = jnp.where(iota == am, big, d)
    idx = jnp.concatenate(picks, axis=1)
    out_ref[0] = idx + b * n_total


def _make_sc_gather(n_rows, d_model):
    NC, NS = 2, 16
    NW = NC * NS
    per_w = n_rows // NW
    CH = 128
    iters = per_w // CH
    mesh = plsc.VectorSubcoreMesh(core_axis_name="c", subcore_axis_name="s")

    @functools.partial(
        pl.kernel,
        out_type=jax.ShapeDtypeStruct((n_rows, d_model), jnp.float32),
        mesh=mesh,
        scratch_types=[
            pltpu.VMEM((CH,), jnp.int32),
            pltpu.VMEM((CH, d_model), jnp.float32),
            pltpu.SemaphoreType.DMA,
        ],
    )
    def gather(table_hbm, idx_hbm, out_hbm, idx_v, rows_v, sem):
        wid = lax.axis_index("s") * NC + lax.axis_index("c")
        base = wid * per_w

        def body(i, carry):
            off = pl.multiple_of(base + i * CH, CH)
            pltpu.sync_copy(idx_hbm.at[pl.ds(off, CH)], idx_v)
            pltpu.async_copy(table_hbm.at[idx_v], rows_v, sem).wait()
            pltpu.sync_copy(rows_v, out_hbm.at[pl.ds(off, CH)])
            return carry

        lax.fori_loop(0, iters, body, 0)

    return gather


def _conv_l1(u, f1, wl1T):
    bmT = wl1T[64:, :] - wl1T[:64, :]
    v = jnp.dot(f1, bmT, preferred_element_type=jnp.float32)
    r = f1.shape[0]
    return u.reshape(r, K, 128) + v[:, None, :]


def _stats_body(nbr_ref, f1_ref, wl1T_ref, out_ref):
    i = pl.program_id(1)
    y = _conv_l1(nbr_ref[0], f1_ref[0], wl1T_ref[...])
    r = f1_ref.shape[1]
    y2 = y.reshape(r * K, 128)
    s = jnp.sum(y2, axis=0, keepdims=True)
    ss = jnp.sum(y2 * y2, axis=0, keepdims=True)
    upd = jnp.concatenate([s, ss, jnp.zeros((6, 128), jnp.float32)], axis=0)

    @pl.when(i == 0)
    def _():
        out_ref[0] = jnp.zeros_like(out_ref[0])

    out_ref[0] = out_ref[0] + upd


def _mlp1_body(
    nbr_ref,
    f1_ref,
    wl1T_ref,
    stats_ref,
    g_ref,
    bgn_ref,
    w1aT_ref,
    b1a_ref,
    w1bT_ref,
    b1b_ref,
    lb_ref,
    gmax_ref,
    *,
    n_total,
):
    i = pl.program_id(1)
    y = _conv_l1(nbr_ref[0], f1_ref[0], wl1T_ref[...])
    r = f1_ref.shape[1]

    cnt = jnp.float32(32 * n_total * K)
    ri = lax.broadcasted_iota(jnp.int32, (128, 128), 0) // 32
    ci = lax.broadcasted_iota(jnp.int32, (128, 128), 1) // 32
    p = (ri == ci).astype(jnp.float32)
    s = stats_ref[0, 0:1, :]
    ss = stats_ref[0, 1:2, :]
    mean = jnp.dot(s, p, preferred_element_type=jnp.float32) / cnt
    msq = jnp.dot(ss, p, preferred_element_type=jnp.float32) / cnt
    var = msq - mean * mean
    rstd = lax.rsqrt(var + EPS)
    scale = rstd * g_ref[...]
    shift = bgn_ref[...] - mean * scale

    feat = y * scale[None, :, :] + shift[None, :, :]
    feat = jnp.where(feat > 0, feat, 0.2 * feat)

    lb_ref[0] = jnp.max(feat, axis=1)

    f2 = feat.reshape(r * K, 128)
    g1 = jnp.dot(f2, w1aT_ref[...], preferred_element_type=jnp.float32)
    g1 = jnp.maximum(g1 + b1a_ref[...], 0.0)
    g2 = jnp.dot(g1, w1bT_ref[...], preferred_element_type=jnp.float32)
    g2 = jnp.maximum(g2 + b1b_ref[...], 0.0)
    gm = jnp.max(g2.reshape(r, K, 512), axis=1)
    bm = jnp.max(gm, axis=0, keepdims=True)
    bm8 = jnp.broadcast_to(bm, (8, 512))

    @pl.when(i == 0)
    def _():
        gmax_ref[0] = jnp.full_like(gmax_ref[0], -jnp.inf)

    gmax_ref[0] = jnp.maximum(gmax_ref[0], bm8)


def _mlp2_body(
    lb_ref,
    gmax_ref,
    w2alT_ref,
    w2agT_ref,
    b2a_ref,
    w2bT_ref,
    b2b_ref,
    w3aT_ref,
    b3a_ref,
    w3bT_ref,
    b3b_ref,
    lfT_ref,
    gf_ref,
):
    j = pl.program_id(1)
    lb = lb_ref[0]
    gvec = jnp.max(gmax_ref[0], axis=0, keepdims=True)
    gterm = jnp.dot(gvec, w2agT_ref[...], preferred_element_type=jnp.float32)
    h = jnp.dot(lb, w2alT_ref[...], preferred_element_type=jnp.float32)
    h = jnp.maximum(h + gterm + b2a_ref[...], 0.0)
    lf = jnp.dot(h, w2bT_ref[...], preferred_element_type=jnp.float32)
    lf = jnp.maximum(lf + b2b_ref[...], 0.0)
    lfT_ref[0] = lf.T

    g3 = jnp.dot(lf, w3aT_ref[...], preferred_element_type=jnp.float32)
    g3 = jnp.maximum(g3 + b3a_ref[...], 0.0)
    g4 = jnp.dot(g3, w3bT_ref[...], preferred_element_type=jnp.float32)
    g4 = jnp.maximum(g4 + b3b_ref[...], 0.0)
    bm = jnp.max(g4, axis=0, keepdims=True)
    bm8 = jnp.broadcast_to(bm, (8, 512))

    @pl.when(j == 0)
    def _():
        gf_ref[0] = jnp.full_like(gf_ref[0], -jnp.inf)

    gf_ref[0] = jnp.maximum(gf_ref[0], bm8)


def kernel(
    x,
    f,
    W_it,
    b_it,
    W_l1,
    gn_g,
    gn_b,
    W_m1a,
    b_m1a,
    W_m1b,
    b_m1b,
    W_m2a,
    b_m2a,
    W_m2b,
    b_m2b,
    W_m3a,
    b_m3a,
    W_m3b,
    b_m3b,
):
    B, _, N = x.shape
    R = 256
    R2 = 128
    RN = 512

    pad5 = jnp.zeros((B, N, 5), jnp.float32)
    xT = jnp.concatenate([jnp.transpose(x, (0, 2, 1)), pad5], axis=2)
    fT = jnp.concatenate([jnp.transpose(f, (0, 2, 1)), pad5], axis=2)
    witT = jnp.concatenate([W_it.T, jnp.zeros((5, 64), jnp.float32)], axis=0)

    f1T, tab = pl.pallas_call(
        _f1_body,
        grid=(B,),
        in_specs=[
            pl.BlockSpec((1, N, 8), lambda b: (b, 0, 0)),
            pl.BlockSpec((8, 64), lambda b: (0, 0)),
            pl.BlockSpec((1, 64), lambda b: (0, 0)),
            pl.BlockSpec((128, 128), lambda b: (0, 0)),
        ],
        out_specs=[
            pl.BlockSpec((1, N, 64), lambda b: (b, 0, 0)),
            pl.BlockSpec((1, N, 128), lambda b: (b, 0, 0)),
        ],
        out_shape=[
            jax.ShapeDtypeStruct((B, N, 64), jnp.float32),
            jax.ShapeDtypeStruct((B, N, 128), jnp.float32),
        ],
    )(fT, witT, b_it.reshape(1, 64), W_l1.T)

    fnorm = pl.pallas_call(
        _norms_body,
        grid=(B,),
        in_specs=[pl.BlockSpec((1, 3, N), lambda b: (b, 0, 0))],
        out_specs=pl.BlockSpec((1, 1, N), lambda b: (b, 0, 0)),
        out_shape=jax.ShapeDtypeStruct((B, 1, N), jnp.float32),
    )(x)

    idx = pl.pallas_call(
        functools.partial(_topk_body, n_total=N, rows=R),
        grid=(B, N // R),
        in_specs=[
            pl.BlockSpec((1, R, 8), lambda b, i: (b, i, 0)),
            pl.BlockSpec((1, N, 8), lambda b, i: (b, 0, 0)),
            pl.BlockSpec((1, 1, N), lambda b, i: (b, 0, 0)),
        ],
        out_specs=pl.BlockSpec((1, R, K), lambda b, i: (b, i, 0)),
        out_shape=jax.ShapeDtypeStruct((B, N, K), jnp.int32),
    )(xT, xT, fnorm)

    n_rows = B * N * K
    u_flat = _make_sc_gather(n_rows, 128)(
        tab.reshape(B * N, 128), idx.reshape(n_rows)
    )
    u = u_flat.reshape(B, N * K, 128)

    stats = pl.pallas_call(
        _stats_body,
        grid=(B, N // R2),
        in_specs=[
            pl.BlockSpec((1, R2 * K, 128), lambda b, i: (b, i, 0)),
            pl.BlockSpec((1, R2, 64), lambda b, i: (b, i, 0)),
            pl.BlockSpec((128, 128), lambda b, i: (0, 0)),
        ],
        out_specs=pl.BlockSpec((1, 8, 128), lambda b, i: (b, 0, 0)),
        out_shape=jax.ShapeDtypeStruct((B, 8, 128), jnp.float32),
    )(u, f1T, W_l1.T)

    local_base, gmax = pl.pallas_call(
        functools.partial(_mlp1_body, n_total=N),
        grid=(B, N // R2),
        in_specs=[
            pl.BlockSpec((1, R2 * K, 128), lambda b, i: (b, i, 0)),
            pl.BlockSpec((1, R2, 64), lambda b, i: (b, i, 0)),
            pl.BlockSpec((128, 128), lambda b, i: (0, 0)),
            pl.BlockSpec((1, 8, 128), lambda b, i: (b, 0, 0)),
            pl.BlockSpec((1, 128), lambda b, i: (0, 0)),
            pl.BlockSpec((1, 128), lambda b, i: (0, 0)),
            pl.BlockSpec((128, 256), lambda b, i: (0, 0)),
            pl.BlockSpec((1, 256), lambda b, i: (0, 0)),
            pl.BlockSpec((256, 512), lambda b, i: (0, 0)),
            pl.BlockSpec((1, 512), lambda b, i: (0, 0)),
        ],
        out_specs=[
            pl.BlockSpec((1, R2, 128), lambda b, i: (b, i, 0)),
            pl.BlockSpec((1, 8, 512), lambda b, i: (b, 0, 0)),
        ],
        out_shape=[
            jax.ShapeDtypeStruct((B, N, 128), jnp.float32),
            jax.ShapeDtypeStruct((B, 8, 512), jnp.float32),
        ],
    )(
        u,
        f1T,
        W_l1.T,
        stats,
        gn_g.reshape(1, 128),
        gn_b.reshape(1, 128),
        W_m1a.T,
        b_m1a.reshape(1, 256),
        W_m1b.T,
        b_m1b.reshape(1, 512),
    )

    lfT, gf = pl.pallas_call(
        _mlp2_body,
        grid=(B, N // RN),
        in_specs=[
            pl.BlockSpec((1, RN, 128), lambda b, j: (b, j, 0)),
            pl.BlockSpec((1, 8, 512), lambda b, j: (b, 0, 0)),
            pl.BlockSpec((128, 256), lambda b, j: (0, 0)),
            pl.BlockSpec((512, 256), lambda b, j: (0, 0)),
            pl.BlockSpec((1, 256), lambda b, j: (0, 0)),
            pl.BlockSpec((256, 128), lambda b, j: (0, 0)),
            pl.BlockSpec((1, 128), lambda b, j: (0, 0)),
            pl.BlockSpec((128, 256), lambda b, j: (0, 0)),
            pl.BlockSpec((1, 256), lambda b, j: (0, 0)),
            pl.BlockSpec((256, 512), lambda b, j: (0, 0)),
            pl.BlockSpec((1, 512), lambda b, j: (0, 0)),
        ],
        out_specs=[
            pl.BlockSpec((1, 128, RN), lambda b, j: (b, 0, j)),
            pl.BlockSpec((1, 8, 512), lambda b, j: (b, 0, 0)),
        ],
        out_shape=[
            jax.ShapeDtypeStruct((B, 128, N), jnp.float32),
            jax.ShapeDtypeStruct((B, 8, 512), jnp.float32),
        ],
    )(
        local_base,
        gmax,
        W_m2a[:, :128].T,
        W_m2a[:, 128:].T,
        b_m2a.reshape(1, 256),
        W_m2b.T,
        b_m2b.reshape(1, 128),
        W_m3a.T,
        b_m3a.reshape(1, 256),
        W_m3b.T,
        b_m3b.reshape(1, 512),
    )

    global_feat = gf[:, 0, :].reshape(B, 512, 1)
    return (global_feat, lfT)


# speedup vs baseline: 8.3665x; 1.0587x over previous
import functools

import jax
import jax.numpy as jnp
from jax import lax
from jax.experimental import pallas as pl
from jax.experimental.pallas import tpu as pltpu
from jax.experimental.pallas import tpu_sc as plsc

K = 16
EPS = 1e-5


def _norms_body(x_ref, out_ref):
    x0 = x_ref[0, 0:1, :]
    x1 = x_ref[0, 1:2, :]
    x2 = x_ref[0, 2:3, :]
    out_ref[0] = (x0 * x0 + x1 * x1) + x2 * x2


def _f1_body(fT_ref, w_ref, b_ref, wl1T_ref, out_ref, tab_ref):
    f1 = (
        jnp.dot(fT_ref[0], w_ref[...], preferred_element_type=jnp.float32)
        + b_ref[...]
    )
    out_ref[0] = f1
    tab_ref[0] = jnp.dot(
        f1, wl1T_ref[:64, :], preferred_element_type=jnp.float32
    )


def _topk_body(rows_ref, full_ref, fn_ref, out_ref, *, n_total, rows):
    b = pl.program_id(0)
    xr = rows_ref[0]
    xf = full_ref[0]
    ab = lax.dot_general(
        xr, xf, (((1,), (1,)), ((), ())), preferred_element_type=jnp.float32
    )
    c0 = xr[:, 0:1]
    c1 = xr[:, 1:2]
    c2 = xr[:, 2:3]
    rn = (c0 * c0 + c1 * c1) + c2 * c2
    fn = fn_ref[0]
    d = -2.0 * ab + rn + fn

    iota = lax.broadcasted_iota(jnp.int32, (rows, n_total), 1)
    picks = []
    big = jnp.float32(jnp.inf)
    for _ in range(K):
        m = jnp.min(d, axis=1, keepdims=True)
        am = jnp.min(
            jnp.where(d == m, iota, n_total), axis=1, keepdims=True
        )
        picks.append(am)
        d = jnp.where(iota == am, big, d)
    idx = jnp.concatenate(picks, axis=1)
    out_ref[0] = idx + b * n_total


def _make_sc_gather(n_rows, d_model):
    NC, NS = 2, 16
    NW = NC * NS
    per_w = n_rows // NW
    CH = 128
    iters = per_w // CH
    mesh = plsc.VectorSubcoreMesh(core_axis_name="c", subcore_axis_name="s")

    @functools.partial(
        pl.kernel,
        out_type=jax.ShapeDtypeStruct((n_rows, d_model), jnp.float32),
        mesh=mesh,
        scratch_types=[
            pltpu.VMEM((CH,), jnp.int32),
            pltpu.VMEM((CH, d_model), jnp.float32),
            pltpu.SemaphoreType.DMA,
        ],
    )
    def gather(table_hbm, idx_hbm, out_hbm, idx_v, rows_v, sem):
        wid = lax.axis_index("s") * NC + lax.axis_index("c")
        base = wid * per_w

        def body(i, carry):
            off = pl.multiple_of(base + i * CH, CH)
            pltpu.sync_copy(idx_hbm.at[pl.ds(off, CH)], idx_v)
            pltpu.async_copy(table_hbm.at[idx_v], rows_v, sem).wait()
            pltpu.sync_copy(rows_v, out_hbm.at[pl.ds(off, CH)])
            return carry

        lax.fori_loop(0, iters, body, 0)

    return gather


def _conv_l1(u, f1, wl1T):
    bmT = wl1T[64:, :] - wl1T[:64, :]
    v = jnp.dot(f1, bmT, preferred_element_type=jnp.float32)
    r = f1.shape[0]
    return u.reshape(r, K, 128) + v[:, None, :]


def _stats_body(nbr_ref, f1_ref, wl1T_ref, out_ref):
    i = pl.program_id(1)
    y = _conv_l1(nbr_ref[0], f1_ref[0], wl1T_ref[...])
    r = f1_ref.shape[1]
    y2 = y.reshape(r * K, 128)
    s = jnp.sum(y2, axis=0, keepdims=True)
    ss = jnp.sum(y2 * y2, axis=0, keepdims=True)
    upd = jnp.concatenate([s, ss, jnp.zeros((6, 128), jnp.float32)], axis=0)

    @pl.when(i == 0)
    def _():
        out_ref[0] = jnp.zeros_like(out_ref[0])

    out_ref[0] = out_ref[0] + upd


def _mlp1_body(
    nbr_ref,
    f1_ref,
    wl1T_ref,
    stats_ref,
    g_ref,
    bgn_ref,
    w1aT_ref,
    b1a_ref,
    w1bT_ref,
    b1b_ref,
    lb_ref,
    gmax_ref,
    *,
    n_total,
):
    i = pl.program_id(1)
    y = _conv_l1(nbr_ref[0], f1_ref[0], wl1T_ref[...])
    r = f1_ref.shape[1]

    cnt = jnp.float32(32 * n_total * K)
    ri = lax.broadcasted_iota(jnp.int32, (128, 128), 0) // 32
    ci = lax.broadcasted_iota(jnp.int32, (128, 128), 1) // 32
    p = (ri == ci).astype(jnp.float32)
    s = stats_ref[0, 0:1, :]
    ss = stats_ref[0, 1:2, :]
    mean = jnp.dot(s, p, preferred_element_type=jnp.float32) / cnt
    msq = jnp.dot(ss, p, preferred_element_type=jnp.float32) / cnt
    var = msq - mean * mean
    rstd = lax.rsqrt(var + EPS)
    scale = rstd * g_ref[...]
    shift = bgn_ref[...] - mean * scale

    feat = y * scale[None, :, :] + shift[None, :, :]
    feat = jnp.where(feat > 0, feat, 0.2 * feat)

    lb_ref[0] = jnp.max(feat, axis=1)

    f2 = feat.reshape(r * K, 128)
    g1 = jnp.dot(f2, w1aT_ref[...], preferred_element_type=jnp.float32)
    g1 = jnp.maximum(g1 + b1a_ref[...], 0.0)
    g2 = jnp.dot(g1, w1bT_ref[...], preferred_element_type=jnp.float32)
    g2 = jnp.maximum(g2 + b1b_ref[...], 0.0)
    gm = jnp.max(g2.reshape(r, K, 512), axis=1)
    bm = jnp.max(gm, axis=0, keepdims=True)
    bm8 = jnp.broadcast_to(bm, (8, 512))

    @pl.when(i == 0)
    def _():
        gmax_ref[0] = jnp.full_like(gmax_ref[0], -jnp.inf)

    gmax_ref[0] = jnp.maximum(gmax_ref[0], bm8)


def _mlp2_body(
    lb_ref,
    gmax_ref,
    w2alT_ref,
    w2agT_ref,
    b2a_ref,
    w2bT_ref,
    b2b_ref,
    w3aT_ref,
    b3a_ref,
    w3bT_ref,
    b3b_ref,
    lfT_ref,
    gf_ref,
):
    j = pl.program_id(1)
    lb = lb_ref[0]
    gvec = jnp.max(gmax_ref[0], axis=0, keepdims=True)
    gterm = jnp.dot(gvec, w2agT_ref[...], preferred_element_type=jnp.float32)
    h = jnp.dot(lb, w2alT_ref[...], preferred_element_type=jnp.float32)
    h = jnp.maximum(h + gterm + b2a_ref[...], 0.0)
    lf = jnp.dot(h, w2bT_ref[...], preferred_element_type=jnp.float32)
    lf = jnp.maximum(lf + b2b_ref[...], 0.0)
    lfT_ref[0] = lf.T

    g3 = jnp.dot(lf, w3aT_ref[...], preferred_element_type=jnp.float32)
    g3 = jnp.maximum(g3 + b3a_ref[...], 0.0)
    g4 = jnp.dot(g3, w3bT_ref[...], preferred_element_type=jnp.float32)
    g4 = jnp.maximum(g4 + b3b_ref[...], 0.0)
    bm = jnp.max(g4, axis=0, keepdims=True)
    bm8 = jnp.broadcast_to(bm, (8, 512))

    @pl.when(j == 0)
    def _():
        gf_ref[0] = jnp.full_like(gf_ref[0], -jnp.inf)

    gf_ref[0] = jnp.maximum(gf_ref[0], bm8)


def kernel(
    x,
    f,
    W_it,
    b_it,
    W_l1,
    gn_g,
    gn_b,
    W_m1a,
    b_m1a,
    W_m1b,
    b_m1b,
    W_m2a,
    b_m2a,
    W_m2b,
    b_m2b,
    W_m3a,
    b_m3a,
    W_m3b,
    b_m3b,
):
    B, _, N = x.shape
    R = 256
    R2 = 128
    RN = 512

    pad5 = jnp.zeros((B, N, 5), jnp.float32)
    xT = jnp.concatenate([jnp.transpose(x, (0, 2, 1)), pad5], axis=2)
    fT = jnp.concatenate([jnp.transpose(f, (0, 2, 1)), pad5], axis=2)
    witT = jnp.concatenate([W_it.T, jnp.zeros((5, 64), jnp.float32)], axis=0)

    f1T, tab = pl.pallas_call(
        _f1_body,
        grid=(B,),
        in_specs=[
            pl.BlockSpec((1, N, 8), lambda b: (b, 0, 0)),
            pl.BlockSpec((8, 64), lambda b: (0, 0)),
            pl.BlockSpec((1, 64), lambda b: (0, 0)),
            pl.BlockSpec((128, 128), lambda b: (0, 0)),
        ],
        out_specs=[
            pl.BlockSpec((1, N, 64), lambda b: (b, 0, 0)),
            pl.BlockSpec((1, N, 128), lambda b: (b, 0, 0)),
        ],
        out_shape=[
            jax.ShapeDtypeStruct((B, N, 64), jnp.float32),
            jax.ShapeDtypeStruct((B, N, 128), jnp.float32),
        ],
    )(fT, witT, b_it.reshape(1, 64), W_l1.T)

    fnorm = pl.pallas_call(
        _norms_body,
        grid=(B,),
        in_specs=[pl.BlockSpec((1, 3, N), lambda b: (b, 0, 0))],
        out_specs=pl.BlockSpec((1, 1, N), lambda b: (b, 0, 0)),
        out_shape=jax.ShapeDtypeStruct((B, 1, N), jnp.float32),
    )(x)

    gather = _make_sc_gather(N * K, 128)

    def topk_b(b):
        return pl.pallas_call(
            functools.partial(_topk_body, n_total=N, rows=R),
            grid=(1, N // R),
            in_specs=[
                pl.BlockSpec((1, R, 8), lambda z, i: (z, i, 0)),
                pl.BlockSpec((1, N, 8), lambda z, i: (z, 0, 0)),
                pl.BlockSpec((1, 1, N), lambda z, i: (z, 0, 0)),
            ],
            out_specs=pl.BlockSpec((1, R, K), lambda z, i: (z, i, 0)),
            out_shape=jax.ShapeDtypeStruct((1, N, K), jnp.int32),
        )(xT[b : b + 1], xT[b : b + 1], fnorm[b : b + 1])

    def stats_b(u, f1b):
        return pl.pallas_call(
            _stats_body,
            grid=(1, N // R2),
            in_specs=[
                pl.BlockSpec((1, R2 * K, 128), lambda z, i: (z, i, 0)),
                pl.BlockSpec((1, R2, 64), lambda z, i: (z, i, 0)),
                pl.BlockSpec((128, 128), lambda z, i: (0, 0)),
            ],
            out_specs=pl.BlockSpec((1, 8, 128), lambda z, i: (z, 0, 0)),
            out_shape=jax.ShapeDtypeStruct((1, 8, 128), jnp.float32),
        )(u, f1b, W_l1.T)

    def mlp1_b(u, f1b, stats):
        return pl.pallas_call(
            functools.partial(_mlp1_body, n_total=N),
            grid=(1, N // R2),
            in_specs=[
                pl.BlockSpec((1, R2 * K, 128), lambda z, i: (z, i, 0)),
                pl.BlockSpec((1, R2, 64), lambda z, i: (z, i, 0)),
                pl.BlockSpec((128, 128), lambda z, i: (0, 0)),
                pl.BlockSpec((1, 8, 128), lambda z, i: (z, 0, 0)),
                pl.BlockSpec((1, 128), lambda z, i: (0, 0)),
                pl.BlockSpec((1, 128), lambda z, i: (0, 0)),
                pl.BlockSpec((128, 256), lambda z, i: (0, 0)),
                pl.BlockSpec((1, 256), lambda z, i: (0, 0)),
                pl.BlockSpec((256, 512), lambda z, i: (0, 0)),
                pl.BlockSpec((1, 512), lambda z, i: (0, 0)),
            ],
            out_specs=[
                pl.BlockSpec((1, R2, 128), lambda z, i: (z, i, 0)),
                pl.BlockSpec((1, 8, 512), lambda z, i: (z, 0, 0)),
            ],
            out_shape=[
                jax.ShapeDtypeStruct((1, N, 128), jnp.float32),
                jax.ShapeDtypeStruct((1, 8, 512), jnp.float32),
            ],
        )(
            u,
            f1b,
            W_l1.T,
            stats,
            gn_g.reshape(1, 128),
            gn_b.reshape(1, 128),
            W_m1a.T,
            b_m1a.reshape(1, 256),
            W_m1b.T,
            b_m1b.reshape(1, 512),
        )

    def mlp2_b(local_base, gmax):
        return pl.pallas_call(
            _mlp2_body,
            grid=(1, N // RN),
            in_specs=[
                pl.BlockSpec((1, RN, 128), lambda z, j: (z, j, 0)),
                pl.BlockSpec((1, 8, 512), lambda z, j: (z, 0, 0)),
                pl.BlockSpec((128, 256), lambda z, j: (0, 0)),
                pl.BlockSpec((512, 256), lambda z, j: (0, 0)),
                pl.BlockSpec((1, 256), lambda z, j: (0, 0)),
                pl.BlockSpec((256, 128), lambda z, j: (0, 0)),
                pl.BlockSpec((1, 128), lambda z, j: (0, 0)),
                pl.BlockSpec((128, 256), lambda z, j: (0, 0)),
                pl.BlockSpec((1, 256), lambda z, j: (0, 0)),
                pl.BlockSpec((256, 512), lambda z, j: (0, 0)),
                pl.BlockSpec((1, 512), lambda z, j: (0, 0)),
            ],
            out_specs=[
                pl.BlockSpec((1, 128, RN), lambda z, j: (z, 0, j)),
                pl.BlockSpec((1, 8, 512), lambda z, j: (z, 0, 0)),
            ],
            out_shape=[
                jax.ShapeDtypeStruct((1, 128, N), jnp.float32),
                jax.ShapeDtypeStruct((1, 8, 512), jnp.float32),
            ],
        )(
            local_base,
            gmax,
            W_m2a[:, :128].T,
            W_m2a[:, 128:].T,
            b_m2a.reshape(1, 256),
            W_m2b.T,
            b_m2b.reshape(1, 128),
            W_m3a.T,
            b_m3a.reshape(1, 256),
            W_m3b.T,
            b_m3b.reshape(1, 512),
        )

    lfTs, gfs = [], []
    for b in range(B):
        idx = topk_b(b)
        u = gather(tab[b], idx.reshape(N * K)).reshape(1, N * K, 128)
        f1b = f1T[b : b + 1]
        st = stats_b(u, f1b)
        local_base, gmax = mlp1_b(u, f1b, st)
        lfT_b, gf_b = mlp2_b(local_base, gmax)
        lfTs.append(lfT_b)
        gfs.append(gf_b[:, 0, :])

    lfT = jnp.concatenate(lfTs, axis=0)
    global_feat = jnp.concatenate(gfs, axis=0).reshape(B, 512, 1)
    return (global_feat, lfT)
